# baseline (speedup 1.0000x reference)
import functools

import jax
import jax.numpy as jnp
from jax import lax
from jax.experimental import pallas as pl
from jax.experimental.pallas import tpu as pltpu
from jax.experimental.pallas import tpu_sc as plsc

_NUM_USERS = 100000
_MAX_MEM = 16
_D = 64
_TOP_K = 4
_INV_TEMP = 10.0
_BATCH = 4096

_NC = 2
_NS = 16
_NW = _NC * _NS
_BPW = _BATCH // _NW
_CH = 32
_NCHUNK = _BPW // _CH


def _fast_rsqrt(x):
    i = plsc.bitcast(x, jnp.int32)
    i = jnp.int32(0x5F3759DF) - lax.shift_right_logical(i, 1)
    r = plsc.bitcast(i, jnp.float32)
    for _ in range(3):
        r = r * (1.5 - 0.5 * x * r * r)
    return r


def _retrieve_body(qn_hbm, keys_hbm, vals_hbm, uid_hbm, cnt_hbm, out_hbm,
                   uid_v, cnt_v, qn_v, keys_v, vals_v, out_v, sem, sem2):
    wid = lax.axis_index("s") * _NC + lax.axis_index("c")
    base = wid * _BPW

    pltpu.sync_copy(uid_hbm.at[pl.ds(base, _BPW)], uid_v)
    pltpu.sync_copy(qn_hbm.at[pl.ds(base * _D, _BPW * _D)], qn_v)
    pltpu.async_copy(cnt_hbm.at[uid_v], cnt_v, sem).wait()

    iota = lax.iota(jnp.int32, 16)
    first4 = iota < _TOP_K

    for ci in range(_NCHUNK):
        idx_ref = uid_v.at[pl.ds(ci * _CH, _CH)]
        kcp = pltpu.async_copy(keys_hbm.at[idx_ref], keys_v, sem)
        vcp = pltpu.async_copy(vals_hbm.at[idx_ref], vals_v, sem2)
        kcp.wait()
        vcp.wait()

        def q_body(q, carry, ci=ci):
            qq = ci * _CH + q
            rows = jnp.broadcast_to(q, (16,)).astype(jnp.int32)
            qsplat = jnp.broadcast_to(qq, (16,)).astype(jnp.int32)
            dot = jnp.zeros((16,), jnp.float32)
            nrm = jnp.zeros((16,), jnp.float32)
            qoff = qq * _D
            for r in range(_D // 16):
                qblk = qn_v[pl.ds(qoff + 16 * r, 16)]
                for j in range(16):
                    d = 16 * r + j
                    kcol = plsc.load_gather(
                        keys_v, [rows, iota, jnp.full((16,), d, jnp.int32)])
                    dot = dot + kcol * qblk[j]
                    nrm = nrm + kcol * kcol
            cntv = plsc.load_gather(cnt_v, [qsplat])
            sims = dot * _fast_rsqrt(jnp.maximum(nrm, 1e-24))
            msims = jnp.where(iota < cntv, sims, jnp.float32(-1e9))
            vmax = jnp.max(msims)
            sk, sv = plsc.sort_key_val(msims, iota, descending=True)
            e = jnp.where(first4, jnp.exp((sk - vmax) * _INV_TEMP), 0.0)
            w = e / jnp.sum(e)
            accs = [jnp.zeros((16,), jnp.float32) for _ in range(_D // 16)]
            for k in range(_TOP_K):
                slot = jnp.broadcast_to(sv[k], (16,)).astype(jnp.int32)
                wk = w[k]
                for r in range(_D // 16):
                    vrow = plsc.load_gather(vals_v, [rows, slot, iota + 16 * r])
                    accs[r] = accs[r] + wk * vrow
            for r in range(_D // 16):
                out_v[pl.ds(q * _D + 16 * r, 16)] = accs[r]
            return carry

        lax.fori_loop(0, _CH, q_body, 0)

        pltpu.sync_copy(out_v,
                        out_hbm.at[pl.ds((base + ci * _CH) * _D, _CH * _D)])


_retrieve = functools.partial(
    pl.kernel,
    out_type=jax.ShapeDtypeStruct((_BATCH * _D,), jnp.float32),
    mesh=plsc.VectorSubcoreMesh(core_axis_name="c", subcore_axis_name="s"),
    compiler_params=pltpu.CompilerParams(use_tc_tiling_on_sc=False,
                                         needs_layout_passes=False),
    scratch_types=[
        pltpu.VMEM((_BPW,), jnp.int32),
        pltpu.VMEM((_BPW,), jnp.int32),
        pltpu.VMEM((_BPW * _D,), jnp.float32),
        pltpu.VMEM((_CH, _MAX_MEM, _D), jnp.float32),
        pltpu.VMEM((_CH, _MAX_MEM, _D), jnp.float32),
        pltpu.VMEM((_CH * _D,), jnp.float32),
        pltpu.SemaphoreType.DMA,
        pltpu.SemaphoreType.DMA,
    ],
)(_retrieve_body)


def _qn_body(q_ref, wk_ref, o_ref):
    y = lax.dot_general(q_ref[...], wk_ref[...], (((1,), (1,)), ((), ())),
                        preferred_element_type=jnp.float32)
    n = jnp.sqrt(jnp.sum(y * y, axis=-1, keepdims=True))
    o_ref[...] = y / jnp.maximum(n, 1e-12)


_qn_call = pl.pallas_call(
    _qn_body,
    out_shape=jax.ShapeDtypeStruct((_BATCH, _D), jnp.float32),
)


def _proj_body(b_ref, wv_ref, o_ref):
    o_ref[...] = lax.dot_general(b_ref[...], wv_ref[...],
                                 (((1,), (1,)), ((), ())),
                                 preferred_element_type=jnp.float32)


_proj_call = pl.pallas_call(
    _proj_body,
    out_shape=jax.ShapeDtypeStruct((_BATCH, _D), jnp.float32),
)


def kernel(query, keys_buf, values_buf, W_key, W_val, episodic_scale,
           user_ids, memory_count):
    qn = _qn_call(query, W_key)
    uid = user_ids.astype(jnp.int32)
    cnt = memory_count.astype(jnp.int32)
    blended = _retrieve(qn.reshape(-1), keys_buf, values_buf, uid, cnt)
    blended = blended.reshape(_BATCH, _D)
    return _proj_call(blended, W_val * episodic_scale)


# speedup vs baseline: 2.2080x; 2.2080x over previous
import functools

import jax
import jax.numpy as jnp
from jax import lax
from jax.experimental import pallas as pl
from jax.experimental.pallas import tpu as pltpu
from jax.experimental.pallas import tpu_sc as plsc

_NUM_USERS = 100000
_MAX_MEM = 16
_D = 64
_TOP_K = 4
_INV_TEMP = 10.0
_BATCH = 4096

_NC = 2
_NS = 16
_NW = _NC * _NS
_BPW = _BATCH // _NW
_CH = 32
_NCHUNK = _BPW // _CH


def _fast_rsqrt(x):
    i = plsc.bitcast(x, jnp.int32)
    i = jnp.int32(0x5F3759DF) - lax.shift_right_logical(i, 1)
    r = plsc.bitcast(i, jnp.float32)
    for _ in range(3):
        r = r * (1.5 - 0.5 * x * r * r)
    return r


def _retrieve_body(qn_hbm, keys_hbm, vals_hbm, uid_hbm, cnt_hbm, out_hbm,
                   uid_v, cnt_v, qn_v, keys_v, vals_v, out_v, sem, sem2):
    wid = lax.axis_index("s") * _NC + lax.axis_index("c")
    base = wid * _BPW

    pltpu.sync_copy(uid_hbm.at[pl.ds(base, _BPW)], uid_v)
    pltpu.sync_copy(qn_hbm.at[pl.ds(base * _D, _BPW * _D)], qn_v)
    pltpu.async_copy(cnt_hbm.at[uid_v], cnt_v, sem).wait()

    iota = lax.iota(jnp.int32, 16)
    first4 = iota < _TOP_K

    for ci in range(_NCHUNK):
        idx_ref = uid_v.at[pl.ds(ci * _CH, _CH)]
        kcp = pltpu.async_copy(keys_hbm.at[idx_ref], keys_v, sem)
        vcp = pltpu.async_copy(vals_hbm.at[idx_ref], vals_v, sem2)
        kcp.wait()
        vcp.wait()

        def q_body(q, carry, ci=ci):
            qq = ci * _CH + q
            rows = jnp.broadcast_to(q, (16,)).astype(jnp.int32)
            qsplat = jnp.broadcast_to(qq, (16,)).astype(jnp.int32)
            dot = jnp.zeros((16,), jnp.float32)
            nrm = jnp.zeros((16,), jnp.float32)
            qoff = qq * _D
            for r in range(_D // 16):
                qblk = qn_v[pl.ds(qoff + 16 * r, 16)]
                for j in range(16):
                    off = iota * _D + (16 * r + j)
                    kcol = plsc.load_gather(
                        keys_v, [rows, lax.shift_right_logical(off, 7),
                                 jnp.bitwise_and(off, 127)])
                    dot = dot + kcol * qblk[j]
                    nrm = nrm + kcol * kcol
            cntv = plsc.load_gather(cnt_v, [qsplat])
            sims = dot * _fast_rsqrt(jnp.maximum(nrm, 1e-24))
            msims = jnp.where(iota < cntv, sims, jnp.float32(-1e9))
            vmax = jnp.max(msims)
            sk, sv = plsc.sort_key_val(msims, iota, descending=True)
            e = jnp.where(first4, jnp.exp((sk - vmax) * _INV_TEMP), 0.0)
            w = e / jnp.sum(e)
            accs = [jnp.zeros((16,), jnp.float32) for _ in range(_D // 16)]
            for k in range(_TOP_K):
                slot = jnp.broadcast_to(sv[k], (16,)).astype(jnp.int32)
                wk = w[k]
                for r in range(_D // 16):
                    off = slot * _D + (16 * r) + iota
                    vrow = plsc.load_gather(
                        vals_v, [rows, lax.shift_right_logical(off, 7),
                                 jnp.bitwise_and(off, 127)])
                    accs[r] = accs[r] + wk * vrow
            for r in range(_D // 16):
                out_v[pl.ds(q * _D + 16 * r, 16)] = accs[r]
            return carry

        lax.fori_loop(0, _CH, q_body, 0)

        pltpu.sync_copy(out_v,
                        out_hbm.at[pl.ds((base + ci * _CH) * _D, _CH * _D)])


_retrieve = functools.partial(
    pl.kernel,
    out_type=jax.ShapeDtypeStruct((_BATCH * _D,), jnp.float32),
    mesh=plsc.VectorSubcoreMesh(core_axis_name="c", subcore_axis_name="s"),
    compiler_params=pltpu.CompilerParams(use_tc_tiling_on_sc=True,
                                         needs_layout_passes=False),
    scratch_types=[
        pltpu.VMEM((_BPW,), jnp.int32),
        pltpu.VMEM((_BPW,), jnp.int32),
        pltpu.VMEM((_BPW * _D,), jnp.float32),
        pltpu.VMEM((_CH, 8, 128), jnp.float32),
        pltpu.VMEM((_CH, 8, 128), jnp.float32),
        pltpu.VMEM((_CH * _D,), jnp.float32),
        pltpu.SemaphoreType.DMA,
        pltpu.SemaphoreType.DMA,
    ],
)(_retrieve_body)


def _qn_body(q_ref, wk_ref, o_ref):
    y = lax.dot_general(q_ref[...], wk_ref[...], (((1,), (1,)), ((), ())),
                        preferred_element_type=jnp.float32)
    n = jnp.sqrt(jnp.sum(y * y, axis=-1, keepdims=True))
    o_ref[...] = y / jnp.maximum(n, 1e-12)


_qn_call = pl.pallas_call(
    _qn_body,
    out_shape=jax.ShapeDtypeStruct((_BATCH, _D), jnp.float32),
)


def _proj_body(b_ref, wv_ref, o_ref):
    o_ref[...] = lax.dot_general(b_ref[...], wv_ref[...],
                                 (((1,), (1,)), ((), ())),
                                 preferred_element_type=jnp.float32)


_proj_call = pl.pallas_call(
    _proj_body,
    out_shape=jax.ShapeDtypeStruct((_BATCH, _D), jnp.float32),
)


def kernel(query, keys_buf, values_buf, W_key, W_val, episodic_scale,
           user_ids, memory_count):
    qn = _qn_call(query, W_key)
    uid = user_ids.astype(jnp.int32)
    cnt = memory_count.astype(jnp.int32)
    keys3 = keys_buf.reshape(_NUM_USERS, 8, 128)
    vals3 = values_buf.reshape(_NUM_USERS, 8, 128)
    blended = _retrieve(qn.reshape(-1), keys3, vals3, uid, cnt)
    blended = blended.reshape(_BATCH, _D)
    return _proj_call(blended, W_val * episodic_scale)
